# baseline (speedup 1.0000x reference)
import functools

import jax
import jax.numpy as jnp
from jax import lax
from jax.experimental import pallas as pl
from jax.experimental.pallas import tpu as pltpu
from jax.experimental.pallas import tpu_sc as plsc

_NUM_ROWS = 128
_ROW_DIM = 3
_LANES = 16


def _broadcast_lane0(v):
    zeros = jnp.zeros((_LANES,), jnp.int32)
    dnums = lax.GatherDimensionNumbers(
        offset_dims=(), collapsed_slice_dims=(0,), start_index_map=(0,))
    return lax.gather(
        v, zeros[:, None], dnums, (1,),
        mode=lax.GatherScatterMode.PROMISE_IN_BOUNDS)


def _sc_lookup(view_ids, rot_weight, trans_weight):
    mesh = plsc.VectorSubcoreMesh(core_axis_name="c", subcore_axis_name="s")

    @functools.partial(
        pl.kernel,
        mesh=mesh,
        out_type=(
            jax.ShapeDtypeStruct((1, _ROW_DIM), jnp.float32),
            jax.ShapeDtypeStruct((1, _ROW_DIM), jnp.float32),
        ),
        scratch_types=[
            pltpu.VMEM((_LANES,), jnp.int32),
            pltpu.VMEM((_NUM_ROWS, _ROW_DIM), jnp.float32),
            pltpu.VMEM((_LANES,), jnp.float32),
            pltpu.SemaphoreType.DMA,
        ],
        compiler_params=pltpu.CompilerParams(
            use_tc_tiling_on_sc=False, needs_layout_passes=False),
    )
    def body(idx_hbm, rot_hbm, trans_hbm, theta_hbm, rho_hbm,
             idx_v, tab_v, row_v, sem):
        cid = lax.axis_index("c")
        sid = lax.axis_index("s")

        def lookup(tab_hbm, out_hbm):
            a = pltpu.async_copy(idx_hbm, idx_v.at[pl.ds(0, 1)], sem)
            b = pltpu.async_copy(tab_hbm, tab_v, sem)
            a.wait()
            b.wait()
            row = _broadcast_lane0(idx_v[...])
            col = jnp.minimum(lax.iota(jnp.int32, _LANES), _ROW_DIM - 1)
            row_v[...] = plsc.load_gather(tab_v, [row, col])
            pltpu.sync_copy(row_v.at[pl.ds(0, _ROW_DIM)], out_hbm.at[0])

        @pl.when(jnp.logical_and(sid == 0, cid == 0))
        def _():
            lookup(rot_hbm, theta_hbm)

        @pl.when(jnp.logical_and(sid == 0, cid == 1))
        def _():
            lookup(trans_hbm, rho_hbm)

    return body(view_ids, rot_weight, trans_weight)


def kernel(view_ids, rot_weight, trans_weight):
    idx = view_ids[:1].astype(jnp.int32)
    theta, rho = _sc_lookup(idx, rot_weight, trans_weight)
    return (theta, rho)


# speedup vs baseline: 1.0645x; 1.0645x over previous
import functools

import jax
import jax.numpy as jnp
from jax import lax
from jax.experimental import pallas as pl
from jax.experimental.pallas import tpu as pltpu
from jax.experimental.pallas import tpu_sc as plsc

_NUM_ROWS = 128
_ROW_DIM = 3
_LANES = 16


def _broadcast_lane0(v):
    zeros = jnp.zeros((_LANES,), jnp.int32)
    dnums = lax.GatherDimensionNumbers(
        offset_dims=(), collapsed_slice_dims=(0,), start_index_map=(0,))
    return lax.gather(
        v, zeros[:, None], dnums, (1,),
        mode=lax.GatherScatterMode.PROMISE_IN_BOUNDS)


def _sc_lookup(view_ids, rot_weight, trans_weight):
    mesh = plsc.VectorSubcoreMesh(
        core_axis_name="c", subcore_axis_name="s", num_cores=1)

    @functools.partial(
        pl.kernel,
        mesh=mesh,
        out_type=(
            jax.ShapeDtypeStruct((1, _ROW_DIM), jnp.float32),
            jax.ShapeDtypeStruct((1, _ROW_DIM), jnp.float32),
        ),
        scratch_types=[
            pltpu.VMEM((_LANES,), jnp.int32),
            pltpu.VMEM((_NUM_ROWS, _ROW_DIM), jnp.float32),
            pltpu.VMEM((_LANES,), jnp.float32),
            pltpu.SemaphoreType.DMA,
        ],
        compiler_params=pltpu.CompilerParams(
            use_tc_tiling_on_sc=False, needs_layout_passes=False),
    )
    def body(idx_hbm, rot_hbm, trans_hbm, theta_hbm, rho_hbm,
             idx_v, tab_v, row_v, sem):
        sid = lax.axis_index("s")

        def lookup(tab_hbm, out_hbm):
            a = pltpu.async_copy(idx_hbm, idx_v.at[pl.ds(0, 1)], sem)
            b = pltpu.async_copy(tab_hbm, tab_v, sem)
            a.wait()
            b.wait()
            row = _broadcast_lane0(idx_v[...])
            col = jnp.minimum(lax.iota(jnp.int32, _LANES), _ROW_DIM - 1)
            row_v[...] = plsc.load_gather(tab_v, [row, col])
            pltpu.sync_copy(row_v.at[pl.ds(0, _ROW_DIM)], out_hbm.at[0])

        @pl.when(sid == 0)
        def _():
            lookup(rot_hbm, theta_hbm)

        @pl.when(sid == 1)
        def _():
            lookup(trans_hbm, rho_hbm)

    return body(view_ids, rot_weight, trans_weight)


def kernel(view_ids, rot_weight, trans_weight):
    idx = view_ids[:1].astype(jnp.int32)
    theta, rho = _sc_lookup(idx, rot_weight, trans_weight)
    return (theta, rho)


# speedup vs baseline: 1.0787x; 1.0133x over previous
import functools

import jax
import jax.numpy as jnp
from jax import lax
from jax.experimental import pallas as pl
from jax.experimental.pallas import tpu as pltpu
from jax.experimental.pallas import tpu_sc as plsc

_NUM_ROWS = 128
_ROW_DIM = 3
_LANES = 16


def _broadcast_lane0(v):
    zeros = jnp.zeros((_LANES,), jnp.int32)
    dnums = lax.GatherDimensionNumbers(
        offset_dims=(), collapsed_slice_dims=(0,), start_index_map=(0,))
    return lax.gather(
        v, zeros[:, None], dnums, (1,),
        mode=lax.GatherScatterMode.PROMISE_IN_BOUNDS)


def _sc_lookup(view_ids, rot_weight, trans_weight):
    mesh = plsc.VectorSubcoreMesh(
        core_axis_name="c", subcore_axis_name="s", num_cores=1,
        num_subcores=2)

    @functools.partial(
        pl.kernel,
        mesh=mesh,
        out_type=(
            jax.ShapeDtypeStruct((1, _ROW_DIM), jnp.float32),
            jax.ShapeDtypeStruct((1, _ROW_DIM), jnp.float32),
        ),
        scratch_types=[
            pltpu.VMEM((_LANES,), jnp.int32),
            pltpu.VMEM((_NUM_ROWS, _ROW_DIM), jnp.float32),
            pltpu.VMEM((_LANES,), jnp.float32),
            pltpu.SemaphoreType.DMA,
        ],
        compiler_params=pltpu.CompilerParams(
            use_tc_tiling_on_sc=False, needs_layout_passes=False),
    )
    def body(idx_hbm, rot_hbm, trans_hbm, theta_hbm, rho_hbm,
             idx_v, tab_v, row_v, sem):
        sid = lax.axis_index("s")

        def lookup(tab_hbm, out_hbm):
            a = pltpu.async_copy(idx_hbm, idx_v.at[pl.ds(0, 1)], sem)
            b = pltpu.async_copy(tab_hbm, tab_v, sem)
            a.wait()
            b.wait()
            row = _broadcast_lane0(idx_v[...])
            col = jnp.minimum(lax.iota(jnp.int32, _LANES), _ROW_DIM - 1)
            row_v[...] = plsc.load_gather(tab_v, [row, col])
            pltpu.sync_copy(row_v.at[pl.ds(0, _ROW_DIM)], out_hbm.at[0])

        @pl.when(sid == 0)
        def _():
            lookup(rot_hbm, theta_hbm)

        @pl.when(sid == 1)
        def _():
            lookup(trans_hbm, rho_hbm)

    return body(view_ids, rot_weight, trans_weight)


def kernel(view_ids, rot_weight, trans_weight):
    idx = view_ids[:1].astype(jnp.int32)
    theta, rho = _sc_lookup(idx, rot_weight, trans_weight)
    return (theta, rho)
